# baseline (speedup 1.0000x reference)
import functools

import jax
import jax.numpy as jnp
from jax import lax
from jax.experimental import pallas as pl
from jax.experimental.pallas import tpu as pltpu
from jax.experimental.pallas import tpu_sc as plsc

N = 10000
NP = 10240
E = 320000
NC = 2
NS = 16
EPT = E // NS
CH = 125
NCHUNK = EPT // CH
RPT = NP // NS
ZROWS = 128
NZ = RPT // ZROWS
F = 128
FH = F // 2
R = 1000

_mesh = plsc.VectorSubcoreMesh(core_axis_name="c", subcore_axis_name="s")


NBUF = 4


@functools.partial(
    pl.kernel,
    mesh=_mesh,
    out_type=jax.ShapeDtypeStruct((NC, NP, FH), jnp.float32),
    scratch_types=[
        pltpu.VMEM((NCHUNK, CH), jnp.int32),
        pltpu.VMEM((NCHUNK, CH), jnp.int32),
        pltpu.VMEM((NBUF, CH, FH), jnp.float32),
        pltpu.VMEM((ZROWS, FH), jnp.float32),
        pltpu.VMEM((ZROWS, 16), jnp.float32),
        pltpu.VMEM((FH,), jnp.float32),
        pltpu.VMEM((16,), jnp.int32),
        pltpu.VMEM_SHARED((NP, FH), jnp.float32),
        pltpu.SemaphoreType.DMA((NBUF,)),
        pltpu.SemaphoreType.DMA((NBUF,)),
    ],
    compiler_params=pltpu.CompilerParams(use_tc_tiling_on_sc=False),
)
def _prop(xs, srcr, dstr, zrows, dinv8, bvec, mode, out,
          src_v, dst_v, rows_v, zbuf, dv_v, bv_v, mode_v, acc, gsem, ssem):
    cid = lax.axis_index("c")
    sid = lax.axis_index("s")
    row0 = sid * RPT
    pltpu.sync_copy(srcr.at[sid], src_v)
    pltpu.sync_copy(dstr.at[sid], dst_v)
    pltpu.sync_copy(bvec.at[cid], bv_v)
    pltpu.sync_copy(mode, mode_v)
    pltpu.sync_copy(zrows, zbuf)
    for t in range(NZ):
        pltpu.sync_copy(zbuf, acc.at[pl.ds(row0 + t * ZROWS, ZROWS)])
    plsc.subcore_barrier()

    for b in range(NBUF):
        pltpu.async_copy(xs.at[cid].at[src_v.at[b]], rows_v.at[b], gsem.at[b])

    @pl.loop(0, NCHUNK // NBUF - 1)
    def _edges(k):
        j0 = k * NBUF
        for b in range(NBUF):
            pltpu.make_async_copy(
                xs.at[cid].at[src_v.at[j0 + b]], rows_v.at[b], gsem.at[b]
            ).wait()
            pltpu.async_copy(rows_v.at[b], acc.at[dst_v.at[j0 + b]],
                             ssem.at[b], add=True)
        for b in range(NBUF):
            pltpu.make_async_copy(rows_v.at[b], acc.at[dst_v.at[j0 + b]],
                                  ssem.at[b]).wait()
            pltpu.async_copy(xs.at[cid].at[src_v.at[j0 + NBUF + b]],
                             rows_v.at[b], gsem.at[b])

    jlast = NCHUNK - NBUF
    tail = []
    for b in range(NBUF):
        pltpu.make_async_copy(
            xs.at[cid].at[src_v.at[jlast + b]], rows_v.at[b], gsem.at[b]
        ).wait()
        tail.append(
            pltpu.async_copy(rows_v.at[b], acc.at[dst_v.at[jlast + b]],
                             ssem.at[b], add=True))
    for b in range(NBUF):
        tail[b].wait()

    plsc.subcore_barrier()
    m = mode_v[:][0]
    for t in range(NZ):
        pltpu.sync_copy(acc.at[pl.ds(row0 + t * ZROWS, ZROWS)], zbuf)
        pltpu.sync_copy(dinv8.at[pl.ds(row0 + t * ZROWS, ZROWS)], dv_v)

        @pl.when(m == 1)
        def _relu_epi():
            @pl.loop(0, ZROWS)
            def _r(r):
                d = dv_v[r, :]
                for c in range(FH // 16):
                    sl = pl.ds(16 * c, 16)
                    v = zbuf[r, sl]
                    zbuf[r, sl] = jnp.maximum(v * d + bv_v[sl], 0.0) * d

        @pl.when(m == 2)
        def _sig_epi():
            @pl.loop(0, ZROWS)
            def _r(r):
                d = dv_v[r, :]
                for c in range(FH // 16):
                    sl = pl.ds(16 * c, 16)
                    v = zbuf[r, sl]
                    zbuf[r, sl] = 1.0 / (1.0 + jnp.exp(-(v * d + bv_v[sl])))

        pltpu.sync_copy(zbuf, out.at[cid, pl.ds(row0 + t * ZROWS, ZROWS)])


NW = NC * NS
DCHUNK = E // NW // CH
DEGW = 16


@functools.partial(
    pl.kernel,
    mesh=_mesh,
    out_type=jax.ShapeDtypeStruct((NC, NP, DEGW), jnp.float32),
    scratch_types=[
        pltpu.VMEM((DCHUNK, CH), jnp.int32),
        pltpu.VMEM((CH, DEGW), jnp.float32),
        pltpu.VMEM((ZROWS, DEGW), jnp.float32),
        pltpu.VMEM_SHARED((NP, DEGW), jnp.float32),
        pltpu.SemaphoreType.DMA((NBUF,)),
    ],
    compiler_params=pltpu.CompilerParams(use_tc_tiling_on_sc=False),
)
def _deg(dstr, ones, zrows, out, dst_v, ones_v, zbuf, acc, ssem):
    cid = lax.axis_index("c")
    sid = lax.axis_index("s")
    wid = cid * NS + sid
    pltpu.sync_copy(dstr.at[wid], dst_v)
    pltpu.sync_copy(ones, ones_v)
    pltpu.sync_copy(zrows, zbuf)
    row0 = sid * RPT
    for t in range(NZ):
        pltpu.sync_copy(zbuf, acc.at[pl.ds(row0 + t * ZROWS, ZROWS)])
    plsc.subcore_barrier()

    @pl.loop(0, DCHUNK // NBUF)
    def _edges(k):
        j0 = k * NBUF
        scatters = [
            pltpu.async_copy(ones_v, acc.at[dst_v.at[j0 + b]], ssem.at[b],
                             add=True)
            for b in range(NBUF)
        ]
        for b in range(NBUF):
            scatters[b].wait()

    plsc.subcore_barrier()
    for t in range(NZ):
        pltpu.sync_copy(acc.at[pl.ds(row0 + t * ZROWS, ZROWS)], zbuf)
        pltpu.sync_copy(zbuf, out.at[cid, pl.ds(row0 + t * ZROWS, ZROWS)])


def _row(w):
    return pl.BlockSpec((R, w), lambda i: (i, 0))


def _row8():
    return pl.BlockSpec((R, 16), lambda i: (i, 0))


def _half():
    return pl.BlockSpec((NC, R, FH), lambda i: (0, i, 0))


def _full(*shape):
    return pl.BlockSpec(shape, lambda i: tuple(0 for _ in shape))


def _tc(body, out_structs, inputs, specs, out_specs):
    return pl.pallas_call(
        body,
        grid=(N // R,),
        in_specs=specs,
        out_specs=out_specs,
        out_shape=out_structs,
    )(*inputs)


def _o(w):
    return jax.ShapeDtypeStruct((N, w), jnp.float32)


def _so():
    return jax.ShapeDtypeStruct((NC, N, FH), jnp.float32)


def _join(p):
    return jnp.concatenate([p[0, :, :], p[1, :, :]], axis=1)


def _store_split(o_ref, v):
    o_ref[0, :, :] = v[:, :FH]
    o_ref[1, :, :] = v[:, FH:]


def _relu(v):
    return jnp.maximum(v, 0.0)


def _t0(degp, x, dinv_o, s0_o):
    deg = degp[0, :, 0:1] + degp[1, :, 0:1]
    dv = lax.rsqrt(jnp.maximum(deg, 1.0))
    dinv_o[...] = jnp.broadcast_to(dv, (R, 16))
    _store_split(s0_o, x[...] * dv)


def _t1(p, dinv, w1, b1, w2, g2_o):
    d = dinv[:, 0:1]
    u = _join(p) * d
    h1 = _relu(jnp.dot(u, w1[...], preferred_element_type=jnp.float32) + b1[...])
    _store_split(g2_o, jnp.dot(h1 * d, w2[...], preferred_element_type=jnp.float32))


def _t3(p, dinv, w31, b31, w32, b32, eps, mu_o, lv_o, s4_o):
    d = dinv[:, 0:1]
    q = _join(p) * d
    mu = jnp.dot(q, w31[...], preferred_element_type=jnp.float32) + b31[...]
    lv = jnp.dot(q, w32[...], preferred_element_type=jnp.float32) + b32[...]
    mu_o[...] = mu
    lv_o[...] = lv
    z = eps[...] * jnp.exp(0.5 * lv) + mu
    _store_split(s4_o, jnp.concatenate([z * d, jnp.zeros_like(z)], axis=1))


def _t4(p64, dinv, w4, b4, s5_o):
    d = dinv[:, 0:1]
    u = p64[0, :, :] * d
    h4 = _relu(jnp.dot(u, w4[...], preferred_element_type=jnp.float32) + b4[...])
    _store_split(s5_o, h4 * d)


def _t5(p, dinv, w5, b5, w6, g6_o):
    d = dinv[:, 0:1]
    u = _join(p) * d
    h5 = _relu(jnp.dot(u, w5[...], preferred_element_type=jnp.float32) + b5[...])
    _store_split(g6_o, jnp.dot(h5 * d, w6[...], preferred_element_type=jnp.float32))


def kernel(x, edge_index, W1, b1, W2, b2, W31, b31, W32, b32, W4, b4, W5, b5, W6, b6):
    src = edge_index[0].reshape(NS, NCHUNK, CH)
    dst = edge_index[1].reshape(NS, NCHUNK, CH)
    dst32 = edge_index[1].reshape(NW, DCHUNK, CH)
    zeros_h = jnp.zeros((ZROWS, FH), jnp.float32)
    zeros16 = jnp.zeros((ZROWS, DEGW), jnp.float32)
    ones16 = jnp.ones((CH, DEGW), jnp.float32)
    eps = jax.random.normal(jax.random.key(42), (N, 64), jnp.float32)
    b1r = b1.reshape(1, -1)
    b31r = b31.reshape(1, -1); b32r = b32.reshape(1, -1)
    b4r = b4.reshape(1, -1); b5r = b5.reshape(1, -1)
    b2h = b2.reshape(NC, FH)
    b6h = b6.reshape(NC, FH)
    bz = jnp.zeros((NC, FH), jnp.float32)
    mode0 = jnp.zeros((16,), jnp.int32)
    mode1 = jnp.full((16,), 1, jnp.int32)
    mode2 = jnp.full((16,), 2, jnp.int32)

    degp = _deg(dst32, ones16, zeros16)
    dinv, s0 = _tc(
        _t0, (jax.ShapeDtypeStruct((NP, 16), jnp.float32), _so()), (degp, x),
        (pl.BlockSpec((NC, R, DEGW), lambda i: (0, i, 0)), _row(128)),
        (_row8(), _half()),
    )
    a1 = _prop(s0, src, dst, zeros_h, dinv, bz, mode0)
    g2 = _tc(
        _t1, _so(), (a1, dinv, W1, b1r, W2),
        (_half(), _row8(), _full(128, 256), _full(1, 256), _full(256, 128)),
        _half(),
    )
    s3 = _prop(g2, src, dst, zeros_h, dinv, b2h, mode1)
    a3 = _prop(s3, src, dst, zeros_h, dinv, bz, mode0)
    mu, log_var, s4 = _tc(
        _t3, (_o(64), _o(64), _so()),
        (a3, dinv, W31, b31r, W32, b32r, eps),
        (_half(), _row8(), _full(128, 64), _full(1, 64), _full(128, 64),
         _full(1, 64), _row(64)),
        (_row(64), _row(64), _half()),
    )
    a4 = _prop(s4, src, dst, zeros_h, dinv, bz, mode0)
    s5 = _tc(
        _t4, _so(), (a4, dinv, W4, b4r),
        (_half(), _row8(), _full(64, 128), _full(1, 128)),
        _half(),
    )
    a5 = _prop(s5, src, dst, zeros_h, dinv, bz, mode0)
    g6 = _tc(
        _t5, _so(), (a5, dinv, W5, b5r, W6),
        (_half(), _row8(), _full(128, 256), _full(1, 256), _full(256, 128)),
        _half(),
    )
    rec_h = _prop(g6, src, dst, zeros_h, dinv, b6h, mode2)
    recon = jnp.concatenate([rec_h[0, :N, :], rec_h[1, :N, :]], axis=1)
    return (recon, mu, log_var)


# speedup vs baseline: 1.0107x; 1.0107x over previous
import functools

import jax
import jax.numpy as jnp
from jax import lax
from jax.experimental import pallas as pl
from jax.experimental.pallas import tpu as pltpu
from jax.experimental.pallas import tpu_sc as plsc

N = 10000
NP = 10240
E = 320000
NC = 2
NS = 16
EPT = E // NS
CH = 100
NCHUNK = EPT // CH
RPT = NP // NS
ZROWS = 128
NZ = RPT // ZROWS
F = 128
FH = F // 2
R = 1000

_mesh = plsc.VectorSubcoreMesh(core_axis_name="c", subcore_axis_name="s")


NBUF = 5


@functools.partial(
    pl.kernel,
    mesh=_mesh,
    out_type=jax.ShapeDtypeStruct((NC, NP, FH), jnp.float32),
    scratch_types=[
        pltpu.VMEM((NCHUNK, CH), jnp.int32),
        pltpu.VMEM((NCHUNK, CH), jnp.int32),
        pltpu.VMEM((NBUF, CH, FH), jnp.float32),
        pltpu.VMEM((ZROWS, FH), jnp.float32),
        pltpu.VMEM((ZROWS, 16), jnp.float32),
        pltpu.VMEM((FH,), jnp.float32),
        pltpu.VMEM((16,), jnp.int32),
        pltpu.VMEM_SHARED((NP, FH), jnp.float32),
        pltpu.SemaphoreType.DMA((NBUF,)),
        pltpu.SemaphoreType.DMA((NBUF,)),
    ],
    compiler_params=pltpu.CompilerParams(use_tc_tiling_on_sc=False),
)
def _prop(xs, srcr, dstr, zrows, dinv8, bvec, mode, out,
          src_v, dst_v, rows_v, zbuf, dv_v, bv_v, mode_v, acc, gsem, ssem):
    cid = lax.axis_index("c")
    sid = lax.axis_index("s")
    row0 = sid * RPT
    pltpu.sync_copy(srcr.at[sid], src_v)
    pltpu.sync_copy(dstr.at[sid], dst_v)
    pltpu.sync_copy(bvec.at[cid], bv_v)
    pltpu.sync_copy(mode, mode_v)
    pltpu.sync_copy(zrows, zbuf)
    for t in range(NZ):
        pltpu.sync_copy(zbuf, acc.at[pl.ds(row0 + t * ZROWS, ZROWS)])
    plsc.subcore_barrier()

    for b in range(NBUF):
        pltpu.async_copy(xs.at[cid].at[src_v.at[b]], rows_v.at[b], gsem.at[b])

    @pl.loop(0, NCHUNK // NBUF - 1)
    def _edges(k):
        j0 = k * NBUF
        for b in range(NBUF):
            pltpu.make_async_copy(
                xs.at[cid].at[src_v.at[j0 + b]], rows_v.at[b], gsem.at[b]
            ).wait()
            pltpu.async_copy(rows_v.at[b], acc.at[dst_v.at[j0 + b]],
                             ssem.at[b], add=True)
        for b in range(NBUF):
            pltpu.make_async_copy(rows_v.at[b], acc.at[dst_v.at[j0 + b]],
                                  ssem.at[b]).wait()
            pltpu.async_copy(xs.at[cid].at[src_v.at[j0 + NBUF + b]],
                             rows_v.at[b], gsem.at[b])

    jlast = NCHUNK - NBUF
    tail = []
    for b in range(NBUF):
        pltpu.make_async_copy(
            xs.at[cid].at[src_v.at[jlast + b]], rows_v.at[b], gsem.at[b]
        ).wait()
        tail.append(
            pltpu.async_copy(rows_v.at[b], acc.at[dst_v.at[jlast + b]],
                             ssem.at[b], add=True))
    for b in range(NBUF):
        tail[b].wait()

    plsc.subcore_barrier()
    m = mode_v[:][0]
    for t in range(NZ):
        pltpu.sync_copy(acc.at[pl.ds(row0 + t * ZROWS, ZROWS)], zbuf)
        pltpu.sync_copy(dinv8.at[pl.ds(row0 + t * ZROWS, ZROWS)], dv_v)

        @pl.when(m == 1)
        def _relu_epi():
            @pl.loop(0, ZROWS)
            def _r(r):
                d = dv_v[r, :]
                for c in range(FH // 16):
                    sl = pl.ds(16 * c, 16)
                    v = zbuf[r, sl]
                    zbuf[r, sl] = jnp.maximum(v * d + bv_v[sl], 0.0) * d

        @pl.when(m == 2)
        def _sig_epi():
            @pl.loop(0, ZROWS)
            def _r(r):
                d = dv_v[r, :]
                for c in range(FH // 16):
                    sl = pl.ds(16 * c, 16)
                    v = zbuf[r, sl]
                    zbuf[r, sl] = 1.0 / (1.0 + jnp.exp(-(v * d + bv_v[sl])))

        pltpu.sync_copy(zbuf, out.at[cid, pl.ds(row0 + t * ZROWS, ZROWS)])


NW = NC * NS
DCHUNK = E // NW // CH
DEGW = 16


@functools.partial(
    pl.kernel,
    mesh=_mesh,
    out_type=jax.ShapeDtypeStruct((NC, NP, DEGW), jnp.float32),
    scratch_types=[
        pltpu.VMEM((DCHUNK, CH), jnp.int32),
        pltpu.VMEM((CH, DEGW), jnp.float32),
        pltpu.VMEM((ZROWS, DEGW), jnp.float32),
        pltpu.VMEM_SHARED((NP, DEGW), jnp.float32),
        pltpu.SemaphoreType.DMA((NBUF,)),
    ],
    compiler_params=pltpu.CompilerParams(use_tc_tiling_on_sc=False),
)
def _deg(dstr, ones, zrows, out, dst_v, ones_v, zbuf, acc, ssem):
    cid = lax.axis_index("c")
    sid = lax.axis_index("s")
    wid = cid * NS + sid
    pltpu.sync_copy(dstr.at[wid], dst_v)
    pltpu.sync_copy(ones, ones_v)
    pltpu.sync_copy(zrows, zbuf)
    row0 = sid * RPT
    for t in range(NZ):
        pltpu.sync_copy(zbuf, acc.at[pl.ds(row0 + t * ZROWS, ZROWS)])
    plsc.subcore_barrier()

    @pl.loop(0, DCHUNK // NBUF)
    def _edges(k):
        j0 = k * NBUF
        scatters = [
            pltpu.async_copy(ones_v, acc.at[dst_v.at[j0 + b]], ssem.at[b],
                             add=True)
            for b in range(NBUF)
        ]
        for b in range(NBUF):
            scatters[b].wait()

    plsc.subcore_barrier()
    for t in range(NZ):
        pltpu.sync_copy(acc.at[pl.ds(row0 + t * ZROWS, ZROWS)], zbuf)
        pltpu.sync_copy(zbuf, out.at[cid, pl.ds(row0 + t * ZROWS, ZROWS)])


def _row(w):
    return pl.BlockSpec((R, w), lambda i: (i, 0))


def _row8():
    return pl.BlockSpec((R, 16), lambda i: (i, 0))


def _half():
    return pl.BlockSpec((NC, R, FH), lambda i: (0, i, 0))


def _full(*shape):
    return pl.BlockSpec(shape, lambda i: tuple(0 for _ in shape))


def _tc(body, out_structs, inputs, specs, out_specs):
    return pl.pallas_call(
        body,
        grid=(N // R,),
        in_specs=specs,
        out_specs=out_specs,
        out_shape=out_structs,
    )(*inputs)


def _o(w):
    return jax.ShapeDtypeStruct((N, w), jnp.float32)


def _so():
    return jax.ShapeDtypeStruct((NC, N, FH), jnp.float32)


def _join(p):
    return jnp.concatenate([p[0, :, :], p[1, :, :]], axis=1)


def _store_split(o_ref, v):
    o_ref[0, :, :] = v[:, :FH]
    o_ref[1, :, :] = v[:, FH:]


def _relu(v):
    return jnp.maximum(v, 0.0)


def _t0(degp, x, dinv_o, s0_o):
    deg = degp[0, :, 0:1] + degp[1, :, 0:1]
    dv = lax.rsqrt(jnp.maximum(deg, 1.0))
    dinv_o[...] = jnp.broadcast_to(dv, (R, 16))
    _store_split(s0_o, x[...] * dv)


def _t1(p, dinv, w1, b1, w2, g2_o):
    d = dinv[:, 0:1]
    u = _join(p) * d
    h1 = _relu(jnp.dot(u, w1[...], preferred_element_type=jnp.float32) + b1[...])
    _store_split(g2_o, jnp.dot(h1 * d, w2[...], preferred_element_type=jnp.float32))


def _t3(p, dinv, w31, b31, w32, b32, eps, mu_o, lv_o, s4_o):
    d = dinv[:, 0:1]
    q = _join(p) * d
    mu = jnp.dot(q, w31[...], preferred_element_type=jnp.float32) + b31[...]
    lv = jnp.dot(q, w32[...], preferred_element_type=jnp.float32) + b32[...]
    mu_o[...] = mu
    lv_o[...] = lv
    z = eps[...] * jnp.exp(0.5 * lv) + mu
    _store_split(s4_o, jnp.concatenate([z * d, jnp.zeros_like(z)], axis=1))


def _t4(p64, dinv, w4, b4, s5_o):
    d = dinv[:, 0:1]
    u = p64[0, :, :] * d
    h4 = _relu(jnp.dot(u, w4[...], preferred_element_type=jnp.float32) + b4[...])
    _store_split(s5_o, h4 * d)


def _t5(p, dinv, w5, b5, w6, g6_o):
    d = dinv[:, 0:1]
    u = _join(p) * d
    h5 = _relu(jnp.dot(u, w5[...], preferred_element_type=jnp.float32) + b5[...])
    _store_split(g6_o, jnp.dot(h5 * d, w6[...], preferred_element_type=jnp.float32))


def kernel(x, edge_index, W1, b1, W2, b2, W31, b31, W32, b32, W4, b4, W5, b5, W6, b6):
    src = edge_index[0].reshape(NS, NCHUNK, CH)
    dst = edge_index[1].reshape(NS, NCHUNK, CH)
    dst32 = edge_index[1].reshape(NW, DCHUNK, CH)
    zeros_h = jnp.zeros((ZROWS, FH), jnp.float32)
    zeros16 = jnp.zeros((ZROWS, DEGW), jnp.float32)
    ones16 = jnp.ones((CH, DEGW), jnp.float32)
    eps = jax.random.normal(jax.random.key(42), (N, 64), jnp.float32)
    b1r = b1.reshape(1, -1)
    b31r = b31.reshape(1, -1); b32r = b32.reshape(1, -1)
    b4r = b4.reshape(1, -1); b5r = b5.reshape(1, -1)
    b2h = b2.reshape(NC, FH)
    b6h = b6.reshape(NC, FH)
    bz = jnp.zeros((NC, FH), jnp.float32)
    mode0 = jnp.zeros((16,), jnp.int32)
    mode1 = jnp.full((16,), 1, jnp.int32)
    mode2 = jnp.full((16,), 2, jnp.int32)

    degp = _deg(dst32, ones16, zeros16)
    dinv, s0 = _tc(
        _t0, (jax.ShapeDtypeStruct((NP, 16), jnp.float32), _so()), (degp, x),
        (pl.BlockSpec((NC, R, DEGW), lambda i: (0, i, 0)), _row(128)),
        (_row8(), _half()),
    )
    a1 = _prop(s0, src, dst, zeros_h, dinv, bz, mode0)
    g2 = _tc(
        _t1, _so(), (a1, dinv, W1, b1r, W2),
        (_half(), _row8(), _full(128, 256), _full(1, 256), _full(256, 128)),
        _half(),
    )
    s3 = _prop(g2, src, dst, zeros_h, dinv, b2h, mode1)
    a3 = _prop(s3, src, dst, zeros_h, dinv, bz, mode0)
    mu, log_var, s4 = _tc(
        _t3, (_o(64), _o(64), _so()),
        (a3, dinv, W31, b31r, W32, b32r, eps),
        (_half(), _row8(), _full(128, 64), _full(1, 64), _full(128, 64),
         _full(1, 64), _row(64)),
        (_row(64), _row(64), _half()),
    )
    a4 = _prop(s4, src, dst, zeros_h, dinv, bz, mode0)
    s5 = _tc(
        _t4, _so(), (a4, dinv, W4, b4r),
        (_half(), _row8(), _full(64, 128), _full(1, 128)),
        _half(),
    )
    a5 = _prop(s5, src, dst, zeros_h, dinv, bz, mode0)
    g6 = _tc(
        _t5, _so(), (a5, dinv, W5, b5r, W6),
        (_half(), _row8(), _full(128, 256), _full(1, 256), _full(256, 128)),
        _half(),
    )
    rec_h = _prop(g6, src, dst, zeros_h, dinv, b6h, mode2)
    recon = jnp.concatenate([rec_h[0, :N, :], rec_h[1, :N, :]], axis=1)
    return (recon, mu, log_var)


# speedup vs baseline: 1.0248x; 1.0140x over previous
import functools

import jax
import jax.numpy as jnp
from jax import lax
from jax.experimental import pallas as pl
from jax.experimental.pallas import tpu as pltpu
from jax.experimental.pallas import tpu_sc as plsc

N = 10000
NP = 10240
E = 320000
NC = 2
NS = 16
EPT = E // NS
CH = 100
NCHUNK = EPT // CH
RPT = NP // NS
ZROWS = 128
NZ = RPT // ZROWS
F = 128
FH = F // 2
R = 1000

_mesh = plsc.VectorSubcoreMesh(core_axis_name="c", subcore_axis_name="s")


NBUF = 5


@functools.partial(
    pl.kernel,
    mesh=_mesh,
    out_type=jax.ShapeDtypeStruct((NC, NP, FH), jnp.float32),
    scratch_types=[
        pltpu.VMEM((NCHUNK, CH), jnp.int32),
        pltpu.VMEM((NCHUNK, CH), jnp.int32),
        pltpu.VMEM((NBUF, CH, FH), jnp.float32),
        pltpu.VMEM((ZROWS, FH), jnp.float32),
        pltpu.VMEM((ZROWS, 16), jnp.float32),
        pltpu.VMEM((FH,), jnp.float32),
        pltpu.VMEM((16,), jnp.int32),
        pltpu.VMEM_SHARED((NP, FH), jnp.float32),
        pltpu.SemaphoreType.DMA((NBUF,)),
        pltpu.SemaphoreType.DMA((NBUF,)),
    ],
    compiler_params=pltpu.CompilerParams(use_tc_tiling_on_sc=False),
)
def _prop(xs, srcr, dstr, zrows, dinv8, bvec, mode, out,
          src_v, dst_v, rows_v, zbuf, dv_v, bv_v, mode_v, acc, gsem, ssem):
    cid = lax.axis_index("c")
    sid = lax.axis_index("s")
    row0 = sid * RPT
    pltpu.sync_copy(srcr.at[sid], src_v)
    pltpu.sync_copy(dstr.at[sid], dst_v)
    pltpu.sync_copy(bvec.at[cid], bv_v)
    pltpu.sync_copy(mode, mode_v)
    pltpu.sync_copy(zrows, zbuf)
    for t in range(NZ):
        pltpu.sync_copy(zbuf, acc.at[pl.ds(row0 + t * ZROWS, ZROWS)])
    plsc.subcore_barrier()

    for b in range(NBUF):
        pltpu.async_copy(xs.at[cid].at[src_v.at[b]], rows_v.at[b], gsem.at[b])

    @pl.loop(0, NCHUNK // NBUF - 1)
    def _edges(k):
        j0 = k * NBUF
        for b in range(NBUF):
            pltpu.make_async_copy(
                xs.at[cid].at[src_v.at[j0 + b]], rows_v.at[b], gsem.at[b]
            ).wait()
            pltpu.async_copy(rows_v.at[b], acc.at[dst_v.at[j0 + b]],
                             ssem.at[b], add=True)
        for b in range(NBUF):
            pltpu.make_async_copy(rows_v.at[b], acc.at[dst_v.at[j0 + b]],
                                  ssem.at[b]).wait()
            pltpu.async_copy(xs.at[cid].at[src_v.at[j0 + NBUF + b]],
                             rows_v.at[b], gsem.at[b])

    jlast = NCHUNK - NBUF
    tail = []
    for b in range(NBUF):
        pltpu.make_async_copy(
            xs.at[cid].at[src_v.at[jlast + b]], rows_v.at[b], gsem.at[b]
        ).wait()
        tail.append(
            pltpu.async_copy(rows_v.at[b], acc.at[dst_v.at[jlast + b]],
                             ssem.at[b], add=True))
    for b in range(NBUF):
        tail[b].wait()

    plsc.subcore_barrier()
    m = mode_v[:][0]
    for t in range(NZ):
        pltpu.sync_copy(acc.at[pl.ds(row0 + t * ZROWS, ZROWS)], zbuf)

        @pl.when(m == 1)
        def _relu_epi():
            pltpu.sync_copy(dinv8.at[pl.ds(row0 + t * ZROWS, ZROWS)], dv_v)

            @pl.loop(0, ZROWS)
            def _r(r):
                d = dv_v[r, :]
                for c in range(FH // 16):
                    sl = pl.ds(16 * c, 16)
                    v = zbuf[r, sl]
                    zbuf[r, sl] = jnp.maximum(v * d + bv_v[sl], 0.0) * d

        @pl.when(m == 2)
        def _sig_epi():
            pltpu.sync_copy(dinv8.at[pl.ds(row0 + t * ZROWS, ZROWS)], dv_v)

            @pl.loop(0, ZROWS)
            def _r(r):
                d = dv_v[r, :]
                for c in range(FH // 16):
                    sl = pl.ds(16 * c, 16)
                    v = zbuf[r, sl]
                    zbuf[r, sl] = 1.0 / (1.0 + jnp.exp(-(v * d + bv_v[sl])))

        pltpu.sync_copy(zbuf, out.at[cid, pl.ds(row0 + t * ZROWS, ZROWS)])


NW = NC * NS
DCHUNK = E // NW // CH
DEGW = 16


@functools.partial(
    pl.kernel,
    mesh=_mesh,
    out_type=jax.ShapeDtypeStruct((NC, NP, DEGW), jnp.float32),
    scratch_types=[
        pltpu.VMEM((DCHUNK, CH), jnp.int32),
        pltpu.VMEM((CH, DEGW), jnp.float32),
        pltpu.VMEM((ZROWS, DEGW), jnp.float32),
        pltpu.VMEM_SHARED((NP, DEGW), jnp.float32),
        pltpu.SemaphoreType.DMA((NBUF,)),
    ],
    compiler_params=pltpu.CompilerParams(use_tc_tiling_on_sc=False),
)
def _deg(dstr, ones, zrows, out, dst_v, ones_v, zbuf, acc, ssem):
    cid = lax.axis_index("c")
    sid = lax.axis_index("s")
    wid = cid * NS + sid
    pltpu.sync_copy(dstr.at[wid], dst_v)
    pltpu.sync_copy(ones, ones_v)
    pltpu.sync_copy(zrows, zbuf)
    row0 = sid * RPT
    for t in range(NZ):
        pltpu.sync_copy(zbuf, acc.at[pl.ds(row0 + t * ZROWS, ZROWS)])
    plsc.subcore_barrier()

    @pl.loop(0, DCHUNK // NBUF)
    def _edges(k):
        j0 = k * NBUF
        scatters = [
            pltpu.async_copy(ones_v, acc.at[dst_v.at[j0 + b]], ssem.at[b],
                             add=True)
            for b in range(NBUF)
        ]
        for b in range(NBUF):
            scatters[b].wait()

    plsc.subcore_barrier()
    for t in range(NZ):
        pltpu.sync_copy(acc.at[pl.ds(row0 + t * ZROWS, ZROWS)], zbuf)
        pltpu.sync_copy(zbuf, out.at[cid, pl.ds(row0 + t * ZROWS, ZROWS)])


def _row(w):
    return pl.BlockSpec((R, w), lambda i: (i, 0))


def _row8():
    return pl.BlockSpec((R, 16), lambda i: (i, 0))


def _half():
    return pl.BlockSpec((NC, R, FH), lambda i: (0, i, 0))


def _full(*shape):
    return pl.BlockSpec(shape, lambda i: tuple(0 for _ in shape))


def _tc(body, out_structs, inputs, specs, out_specs):
    return pl.pallas_call(
        body,
        grid=(N // R,),
        in_specs=specs,
        out_specs=out_specs,
        out_shape=out_structs,
    )(*inputs)


def _o(w):
    return jax.ShapeDtypeStruct((N, w), jnp.float32)


def _so():
    return jax.ShapeDtypeStruct((NC, N, FH), jnp.float32)


def _join(p):
    return jnp.concatenate([p[0, :, :], p[1, :, :]], axis=1)


def _store_split(o_ref, v):
    o_ref[0, :, :] = v[:, :FH]
    o_ref[1, :, :] = v[:, FH:]


def _relu(v):
    return jnp.maximum(v, 0.0)


def _t0(degp, x, dinv_o, s0_o):
    deg = degp[0, :, 0:1] + degp[1, :, 0:1]
    dv = lax.rsqrt(jnp.maximum(deg, 1.0))
    dinv_o[...] = jnp.broadcast_to(dv, (R, 16))
    _store_split(s0_o, x[...] * dv)


def _t1(p, dinv, w1, b1, w2, g2_o):
    d = dinv[:, 0:1]
    u = _join(p) * d
    h1 = _relu(jnp.dot(u, w1[...], preferred_element_type=jnp.float32) + b1[...])
    _store_split(g2_o, jnp.dot(h1 * d, w2[...], preferred_element_type=jnp.float32))


def _t3(p, dinv, w31, b31, w32, b32, eps, mu_o, lv_o, s4_o):
    d = dinv[:, 0:1]
    q = _join(p) * d
    mu = jnp.dot(q, w31[...], preferred_element_type=jnp.float32) + b31[...]
    lv = jnp.dot(q, w32[...], preferred_element_type=jnp.float32) + b32[...]
    mu_o[...] = mu
    lv_o[...] = lv
    z = eps[...] * jnp.exp(0.5 * lv) + mu
    _store_split(s4_o, jnp.concatenate([z * d, jnp.zeros_like(z)], axis=1))


def _t4(p64, dinv, w4, b4, s5_o):
    d = dinv[:, 0:1]
    u = p64[0, :, :] * d
    h4 = _relu(jnp.dot(u, w4[...], preferred_element_type=jnp.float32) + b4[...])
    _store_split(s5_o, h4 * d)


def _t5(p, dinv, w5, b5, w6, g6_o):
    d = dinv[:, 0:1]
    u = _join(p) * d
    h5 = _relu(jnp.dot(u, w5[...], preferred_element_type=jnp.float32) + b5[...])
    _store_split(g6_o, jnp.dot(h5 * d, w6[...], preferred_element_type=jnp.float32))


def kernel(x, edge_index, W1, b1, W2, b2, W31, b31, W32, b32, W4, b4, W5, b5, W6, b6):
    src = edge_index[0].reshape(NS, NCHUNK, CH)
    dst = edge_index[1].reshape(NS, NCHUNK, CH)
    dst32 = edge_index[1].reshape(NW, DCHUNK, CH)
    zeros_h = jnp.zeros((ZROWS, FH), jnp.float32)
    zeros16 = jnp.zeros((ZROWS, DEGW), jnp.float32)
    ones16 = jnp.ones((CH, DEGW), jnp.float32)
    eps = jax.random.normal(jax.random.key(42), (N, 64), jnp.float32)
    b1r = b1.reshape(1, -1)
    b31r = b31.reshape(1, -1); b32r = b32.reshape(1, -1)
    b4r = b4.reshape(1, -1); b5r = b5.reshape(1, -1)
    b2h = b2.reshape(NC, FH)
    b6h = b6.reshape(NC, FH)
    bz = jnp.zeros((NC, FH), jnp.float32)
    mode0 = jnp.zeros((16,), jnp.int32)
    mode1 = jnp.full((16,), 1, jnp.int32)
    mode2 = jnp.full((16,), 2, jnp.int32)

    degp = _deg(dst32, ones16, zeros16)
    dinv, s0 = _tc(
        _t0, (jax.ShapeDtypeStruct((NP, 16), jnp.float32), _so()), (degp, x),
        (pl.BlockSpec((NC, R, DEGW), lambda i: (0, i, 0)), _row(128)),
        (_row8(), _half()),
    )
    a1 = _prop(s0, src, dst, zeros_h, dinv, bz, mode0)
    g2 = _tc(
        _t1, _so(), (a1, dinv, W1, b1r, W2),
        (_half(), _row8(), _full(128, 256), _full(1, 256), _full(256, 128)),
        _half(),
    )
    s3 = _prop(g2, src, dst, zeros_h, dinv, b2h, mode1)
    a3 = _prop(s3, src, dst, zeros_h, dinv, bz, mode0)
    mu, log_var, s4 = _tc(
        _t3, (_o(64), _o(64), _so()),
        (a3, dinv, W31, b31r, W32, b32r, eps),
        (_half(), _row8(), _full(128, 64), _full(1, 64), _full(128, 64),
         _full(1, 64), _row(64)),
        (_row(64), _row(64), _half()),
    )
    a4 = _prop(s4, src, dst, zeros_h, dinv, bz, mode0)
    s5 = _tc(
        _t4, _so(), (a4, dinv, W4, b4r),
        (_half(), _row8(), _full(64, 128), _full(1, 128)),
        _half(),
    )
    a5 = _prop(s5, src, dst, zeros_h, dinv, bz, mode0)
    g6 = _tc(
        _t5, _so(), (a5, dinv, W5, b5r, W6),
        (_half(), _row8(), _full(128, 256), _full(1, 256), _full(256, 128)),
        _half(),
    )
    rec_h = _prop(g6, src, dst, zeros_h, dinv, b6h, mode2)
    recon = jnp.concatenate([rec_h[0, :N, :], rec_h[1, :N, :]], axis=1)
    return (recon, mu, log_var)


# speedup vs baseline: 1.1106x; 1.0837x over previous
import functools

import jax
import jax.numpy as jnp
from jax import lax
from jax.experimental import pallas as pl
from jax.experimental.pallas import tpu as pltpu
from jax.experimental.pallas import tpu_sc as plsc

N = 10000
NP = 10240
E = 320000
NC = 2
NS = 16
EPT = E // NS
CH = 125
NCHUNK = EPT // CH
RPT = NP // NS
ZROWS = 128
NZ = RPT // ZROWS
F = 128
FH = F // 2
R = 1000

_mesh = plsc.VectorSubcoreMesh(core_axis_name="c", subcore_axis_name="s")


NBUF = 5


@functools.partial(
    pl.kernel,
    mesh=_mesh,
    out_type=jax.ShapeDtypeStruct((NC, NP, FH), jnp.float32),
    scratch_types=[
        pltpu.VMEM((NCHUNK, CH), jnp.int32),
        pltpu.VMEM((NCHUNK, CH), jnp.int32),
        pltpu.VMEM((NBUF, CH, FH), jnp.float32),
        pltpu.VMEM((ZROWS, FH), jnp.float32),
        pltpu.VMEM_SHARED((NP, FH), jnp.float32),
        pltpu.SemaphoreType.DMA((NBUF,)),
        pltpu.SemaphoreType.DMA((NBUF,)),
    ],
    compiler_params=pltpu.CompilerParams(use_tc_tiling_on_sc=False),
)
def _prop(xs, srcr, dstr, zrows, out, src_v, dst_v, rows_v, zbuf, acc, gsem, ssem):
    cid = lax.axis_index("c")
    sid = lax.axis_index("s")
    pltpu.sync_copy(srcr.at[sid], src_v)
    pltpu.sync_copy(dstr.at[sid], dst_v)
    pltpu.sync_copy(zrows, zbuf)
    row0 = sid * RPT
    for t in range(NZ):
        pltpu.sync_copy(zbuf, acc.at[pl.ds(row0 + t * ZROWS, ZROWS)])
    plsc.subcore_barrier()

    for b in range(NBUF):
        pltpu.async_copy(xs.at[cid].at[src_v.at[b]], rows_v.at[b], gsem.at[b])

    @pl.loop(0, NCHUNK // NBUF - 1)
    def _edges(k):
        j0 = k * NBUF
        for b in range(NBUF):
            pltpu.make_async_copy(
                xs.at[cid].at[src_v.at[j0 + b]], rows_v.at[b], gsem.at[b]
            ).wait()
            pltpu.async_copy(rows_v.at[b], acc.at[dst_v.at[j0 + b]],
                             ssem.at[b], add=True)
        for b in range(NBUF):
            pltpu.make_async_copy(rows_v.at[b], acc.at[dst_v.at[j0 + b]],
                                  ssem.at[b]).wait()
            pltpu.async_copy(xs.at[cid].at[src_v.at[j0 + NBUF + b]],
                             rows_v.at[b], gsem.at[b])

    jlast = NCHUNK - NBUF
    tail = []
    for b in range(NBUF):
        pltpu.make_async_copy(
            xs.at[cid].at[src_v.at[jlast + b]], rows_v.at[b], gsem.at[b]
        ).wait()
        tail.append(
            pltpu.async_copy(rows_v.at[b], acc.at[dst_v.at[jlast + b]],
                             ssem.at[b], add=True))
    for b in range(NBUF):
        tail[b].wait()

    plsc.subcore_barrier()
    for t in range(NZ):
        pltpu.sync_copy(acc.at[pl.ds(row0 + t * ZROWS, ZROWS)], zbuf)
        pltpu.sync_copy(zbuf, out.at[cid, pl.ds(row0 + t * ZROWS, ZROWS)])


NW = NC * NS
DCHUNK = E // NW // CH
DEGW = 16


@functools.partial(
    pl.kernel,
    mesh=_mesh,
    out_type=jax.ShapeDtypeStruct((NC, NP, DEGW), jnp.float32),
    scratch_types=[
        pltpu.VMEM((DCHUNK, CH), jnp.int32),
        pltpu.VMEM((CH, DEGW), jnp.float32),
        pltpu.VMEM((ZROWS, DEGW), jnp.float32),
        pltpu.VMEM_SHARED((NP, DEGW), jnp.float32),
        pltpu.SemaphoreType.DMA((NBUF,)),
    ],
    compiler_params=pltpu.CompilerParams(use_tc_tiling_on_sc=False),
)
def _deg(dstr, ones, zrows, out, dst_v, ones_v, zbuf, acc, ssem):
    cid = lax.axis_index("c")
    sid = lax.axis_index("s")
    wid = cid * NS + sid
    pltpu.sync_copy(dstr.at[wid], dst_v)
    pltpu.sync_copy(ones, ones_v)
    pltpu.sync_copy(zrows, zbuf)
    row0 = sid * RPT
    for t in range(NZ):
        pltpu.sync_copy(zbuf, acc.at[pl.ds(row0 + t * ZROWS, ZROWS)])
    plsc.subcore_barrier()

    @pl.loop(0, DCHUNK // NBUF)
    def _edges(k):
        j0 = k * NBUF
        scatters = [
            pltpu.async_copy(ones_v, acc.at[dst_v.at[j0 + b]], ssem.at[b],
                             add=True)
            for b in range(NBUF)
        ]
        for b in range(NBUF):
            scatters[b].wait()

    plsc.subcore_barrier()
    for t in range(NZ):
        pltpu.sync_copy(acc.at[pl.ds(row0 + t * ZROWS, ZROWS)], zbuf)
        pltpu.sync_copy(zbuf, out.at[cid, pl.ds(row0 + t * ZROWS, ZROWS)])


def _row(w):
    return pl.BlockSpec((R, w), lambda i: (i, 0))


def _half():
    return pl.BlockSpec((NC, R, FH), lambda i: (0, i, 0))


def _full(*shape):
    return pl.BlockSpec(shape, lambda i: tuple(0 for _ in shape))


def _tc(body, out_structs, inputs, specs, out_specs):
    return pl.pallas_call(
        body,
        grid=(N // R,),
        in_specs=specs,
        out_specs=out_specs,
        out_shape=out_structs,
    )(*inputs)


def _o(w):
    return jax.ShapeDtypeStruct((N, w), jnp.float32)


def _so():
    return jax.ShapeDtypeStruct((NC, N, FH), jnp.float32)


def _join(p):
    return jnp.concatenate([p[0, :, :], p[1, :, :]], axis=1)


def _store_split(o_ref, v):
    o_ref[0, :, :] = v[:, :FH]
    o_ref[1, :, :] = v[:, FH:]


def _relu(v):
    return jnp.maximum(v, 0.0)


def _t0(degp, x, dinv_o, s0_o):
    deg = degp[0, :, 0:1] + degp[1, :, 0:1]
    dv = lax.rsqrt(jnp.maximum(deg, 1.0))
    dinv_o[...] = jnp.broadcast_to(dv, (R, 128))
    _store_split(s0_o, x[...] * dv)


def _t1(p, dinv, w1, b1, w2, g2_o):
    d = dinv[:, 0:1]
    u = _join(p) * d
    h1 = _relu(jnp.dot(u, w1[...], preferred_element_type=jnp.float32) + b1[...])
    _store_split(g2_o, jnp.dot(h1 * d, w2[...], preferred_element_type=jnp.float32))


def _t2(p, dinv, b2, s3_o):
    d = dinv[:, 0:1]
    h2 = _relu(_join(p) * d + b2[...])
    _store_split(s3_o, h2 * d)


def _t3(p, dinv, w31, b31, w32, b32, eps, mu_o, lv_o, s4_o):
    d = dinv[:, 0:1]
    q = _join(p) * d
    mu = jnp.dot(q, w31[...], preferred_element_type=jnp.float32) + b31[...]
    lv = jnp.dot(q, w32[...], preferred_element_type=jnp.float32) + b32[...]
    mu_o[...] = mu
    lv_o[...] = lv
    z = eps[...] * jnp.exp(0.5 * lv) + mu
    _store_split(s4_o, jnp.concatenate([z * d, jnp.zeros_like(z)], axis=1))


def _t4(p64, dinv, w4, b4, s5_o):
    d = dinv[:, 0:1]
    u = p64[0, :, :] * d
    h4 = _relu(jnp.dot(u, w4[...], preferred_element_type=jnp.float32) + b4[...])
    _store_split(s5_o, h4 * d)


def _t5(p, dinv, w5, b5, w6, g6_o):
    d = dinv[:, 0:1]
    u = _join(p) * d
    h5 = _relu(jnp.dot(u, w5[...], preferred_element_type=jnp.float32) + b5[...])
    _store_split(g6_o, jnp.dot(h5 * d, w6[...], preferred_element_type=jnp.float32))


def _t6(p, dinv, b6, recon_o):
    d = dinv[:, 0:1]
    v = _join(p) * d + b6[...]
    recon_o[...] = 1.0 / (1.0 + jnp.exp(-v))


def kernel(x, edge_index, W1, b1, W2, b2, W31, b31, W32, b32, W4, b4, W5, b5, W6, b6):
    src = edge_index[0].reshape(NS, NCHUNK, CH)
    dst = edge_index[1].reshape(NS, NCHUNK, CH)
    dst32 = edge_index[1].reshape(NW, DCHUNK, CH)
    zeros_h = jnp.zeros((ZROWS, FH), jnp.float32)
    zeros16 = jnp.zeros((ZROWS, DEGW), jnp.float32)
    ones16 = jnp.ones((CH, DEGW), jnp.float32)
    eps = jax.random.normal(jax.random.key(42), (N, 64), jnp.float32)
    b1r = b1.reshape(1, -1); b2r = b2.reshape(1, -1)
    b31r = b31.reshape(1, -1); b32r = b32.reshape(1, -1)
    b4r = b4.reshape(1, -1); b5r = b5.reshape(1, -1); b6r = b6.reshape(1, -1)

    degp = _deg(dst32, ones16, zeros16)
    dinv, s0 = _tc(
        _t0, (_o(128), _so()), (degp, x),
        (pl.BlockSpec((NC, R, DEGW), lambda i: (0, i, 0)), _row(128)),
        (_row(128), _half()),
    )
    a1 = _prop(s0, src, dst, zeros_h)
    g2 = _tc(
        _t1, _so(), (a1, dinv, W1, b1r, W2),
        (_half(), _row(128), _full(128, 256), _full(1, 256), _full(256, 128)),
        _half(),
    )
    a2 = _prop(g2, src, dst, zeros_h)
    s3 = _tc(
        _t2, _so(), (a2, dinv, b2r),
        (_half(), _row(128), _full(1, 128)),
        _half(),
    )
    a3 = _prop(s3, src, dst, zeros_h)
    mu, log_var, s4 = _tc(
        _t3, (_o(64), _o(64), _so()),
        (a3, dinv, W31, b31r, W32, b32r, eps),
        (_half(), _row(128), _full(128, 64), _full(1, 64), _full(128, 64),
         _full(1, 64), _row(64)),
        (_row(64), _row(64), _half()),
    )
    a4 = _prop(s4, src, dst, zeros_h)
    s5 = _tc(
        _t4, _so(), (a4, dinv, W4, b4r),
        (_half(), _row(128), _full(64, 128), _full(1, 128)),
        _half(),
    )
    a5 = _prop(s5, src, dst, zeros_h)
    g6 = _tc(
        _t5, _so(), (a5, dinv, W5, b5r, W6),
        (_half(), _row(128), _full(128, 256), _full(1, 256), _full(256, 128)),
        _half(),
    )
    a6 = _prop(g6, src, dst, zeros_h)
    recon = _tc(
        _t6, _o(128), (a6, dinv, b6r),
        (_half(), _row(128), _full(1, 128)),
        _row(128),
    )
    return (recon, mu, log_var)
